# Initial kernel scaffold; baseline (speedup 1.0000x reference)
import jax
import jax.numpy as jnp
from jax.experimental import pallas as pl


def kernel(lang, frames, actions, emb_weight):
    raise NotImplementedError("write your pallas kernel here")


# speedup vs baseline: 3.2275x; 3.2275x over previous
import jax
import jax.numpy as jnp
from jax.experimental import pallas as pl


def _body(lang_ref, frames_ref, actions_ref, emb_ref, out_l, out_f, out_a):
    out_l[...] = lang_ref[...] + emb_ref[0, :][None, :]
    out_f[...] = frames_ref[...] + emb_ref[1, :][None, :]
    out_a[...] = actions_ref[...] + emb_ref[2, :][None, :]


def kernel(lang, frames, actions, emb_weight):
    B, S, D = lang.shape
    R = B * S
    lf = lang.reshape(R, D)
    ff = frames.reshape(R, D)
    af = actions.reshape(R, D)
    BR = 512
    spec = pl.BlockSpec((BR, D), lambda i: (i, 0))
    emb_spec = pl.BlockSpec((3, D), lambda i: (0, 0))
    out = pl.pallas_call(
        _body,
        grid=(R // BR,),
        in_specs=[spec, spec, spec, emb_spec],
        out_specs=[spec, spec, spec],
        out_shape=[jax.ShapeDtypeStruct((R, D), jnp.float32)] * 3,
    )(lf, ff, af, emb_weight)
    return tuple(o.reshape(B, S, D) for o in out)
